# baseline (speedup 1.0000x reference)
import functools

import jax
import jax.numpy as jnp
from jax import lax
from jax.experimental import pallas as pl
from jax.experimental.pallas import tpu as pltpu
from jax.experimental.pallas import tpu_sc as plsc

D = 64
NC, NS = 2, 16
NW = NC * NS
CH = 128


def _sc_gather(user_factors, movie_factors, user_bias, movie_bias,
               uid2d, mid2d, batch):
    bpw = batch // NW
    nch = bpw // CH
    f32 = jnp.float32

    mesh = plsc.VectorSubcoreMesh(core_axis_name="c", subcore_axis_name="s",
                                  num_cores=NC, num_subcores=NS)

    @functools.partial(
        pl.kernel,
        out_type=(
            jax.ShapeDtypeStruct((batch, D), f32),
            jax.ShapeDtypeStruct((batch, D), f32),
            jax.ShapeDtypeStruct((batch,), f32),
            jax.ShapeDtypeStruct((batch,), f32),
        ),
        mesh=mesh,
        compiler_params=pltpu.CompilerParams(use_tc_tiling_on_sc=False),
        scratch_types=[
            pltpu.VMEM((nch, CH), jnp.int32),
            pltpu.VMEM((nch, CH), jnp.int32),
            pltpu.VMEM((bpw, D), f32),
            pltpu.VMEM((bpw, D), f32),
            pltpu.VMEM((bpw,), f32),
            pltpu.VMEM((bpw,), f32),
            pltpu.SemaphoreType.DMA,
        ],
    )
    def body(uf_hbm, mf_hbm, ubt_hbm, mbt_hbm, uid_hbm, mid_hbm,
             uo_hbm, mo_hbm, ubo_hbm, mbo_hbm,
             uidx, midx, urows, mrows, ubv, mbv, sem):
        wid = lax.axis_index("s") * NC + lax.axis_index("c")
        base = wid * bpw
        crow = wid * nch
        pltpu.sync_copy(uid_hbm.at[pl.ds(crow, nch)], uidx)
        pltpu.sync_copy(mid_hbm.at[pl.ds(crow, nch)], midx)
        copies = []
        for j in range(nch):
            sl = pl.ds(j * CH, CH)
            copies.append(pltpu.async_copy(uf_hbm.at[uidx.at[j]], urows.at[sl], sem))
            copies.append(pltpu.async_copy(mf_hbm.at[midx.at[j]], mrows.at[sl], sem))
            copies.append(pltpu.async_copy(ubt_hbm.at[uidx.at[j]], ubv.at[sl], sem))
            copies.append(pltpu.async_copy(mbt_hbm.at[midx.at[j]], mbv.at[sl], sem))
        for c in copies:
            c.wait()
        out_sl = pl.ds(base, bpw)
        pltpu.sync_copy(urows, uo_hbm.at[out_sl])
        pltpu.sync_copy(mrows, mo_hbm.at[out_sl])
        pltpu.sync_copy(ubv, ubo_hbm.at[out_sl])
        pltpu.sync_copy(mbv, mbo_hbm.at[out_sl])

    return body(user_factors, movie_factors, user_bias, movie_bias,
                uid2d, mid2d)


def _ln(x, w, b, eps=1e-5):
    mean = jnp.mean(x, axis=-1, keepdims=True)
    xc = x - mean
    var = jnp.mean(xc * xc, axis=-1, keepdims=True)
    return xc * lax.rsqrt(var + eps) * w + b


def _tc_body(u_ref, m_ref, ub_ref, mb_ref, ids_ref, af_ref,
             unw, unb, mnw, mnb, anw, anb, gb_ref, out_ref):
    blk = u_ref.shape[0]
    u = _ln(jnp.maximum(u_ref[...], 0.0), unw[...], unb[...])
    m = _ln(jnp.maximum(m_ref[...], 0.0), mnw[...], mnb[...])
    a_tab = _ln(jnp.maximum(af_ref[...], 0.0), anw[...], anb[...])
    n_age = af_ref.shape[0]
    onehot = (ids_ref[...] == lax.broadcasted_iota(jnp.int32, (blk, n_age), 1)
              ).astype(jnp.float32)
    ages = jnp.dot(onehot, a_tab, preferred_element_type=jnp.float32)
    dot = jnp.sum(u * m * ages, axis=1, keepdims=True)
    preds = dot * 0.125 + ub_ref[...] + mb_ref[...] + gb_ref[...]
    out_ref[...] = jnp.clip(preds, -0.1, 1.1)


def _tc_compute(u_rows, m_rows, ub, mb, age_ids2d, age_factors,
                unw, unb, mnw, mnb, anw, anb, gb, batch, grid):
    blk = batch // grid
    n_age = age_factors.shape[0]
    row_spec = pl.BlockSpec((blk, D), lambda i: (i, 0))
    col_spec = pl.BlockSpec((blk, 1), lambda i: (i, 0))
    par_spec = pl.BlockSpec((1, D), lambda i: (0, 0))
    return pl.pallas_call(
        _tc_body,
        grid=(grid,),
        in_specs=[
            row_spec, row_spec, col_spec, col_spec, col_spec,
            pl.BlockSpec((n_age, D), lambda i: (0, 0)),
            par_spec, par_spec, par_spec, par_spec, par_spec, par_spec,
            pl.BlockSpec((1, 1), lambda i: (0, 0)),
        ],
        out_specs=col_spec,
        out_shape=jax.ShapeDtypeStruct((batch, 1), jnp.float32),
    )(u_rows, m_rows, ub, mb, age_ids2d, age_factors,
      unw, unb, mnw, mnb, anw, anb, gb)


def kernel(user_ids, movie_ids, age_bucket_ids,
           user_factors, movie_factors, age_factors,
           user_norm_w, user_norm_b, movie_norm_w, movie_norm_b,
           age_norm_w, age_norm_b, user_bias, movie_bias, global_bias):
    batch = user_ids.shape[0]
    uid2d = user_ids.astype(jnp.int32).reshape(batch // CH, CH)
    mid2d = movie_ids.astype(jnp.int32).reshape(batch // CH, CH)
    u_rows, m_rows, ub, mb = _sc_gather(
        user_factors, movie_factors,
        user_bias.reshape(-1), movie_bias.reshape(-1),
        uid2d, mid2d, batch)
    preds = _tc_compute(
        u_rows, m_rows, ub.reshape(batch, 1), mb.reshape(batch, 1),
        age_bucket_ids.astype(jnp.int32).reshape(batch, 1), age_factors,
        user_norm_w.reshape(1, D), user_norm_b.reshape(1, D),
        movie_norm_w.reshape(1, D), movie_norm_b.reshape(1, D),
        age_norm_w.reshape(1, D), age_norm_b.reshape(1, D),
        global_bias.reshape(1, 1), batch, grid=8)
    return preds.reshape(batch)
